# baseline (speedup 1.0000x reference)
import functools

import jax
import jax.numpy as jnp
from jax import lax
from jax.experimental import pallas as pl
from jax.experimental.pallas import tpu as pltpu
from jax.experimental.pallas import tpu_sc as plsc

L = 16
_HI_MASK = -65536


def _entity_pool_sc(table, idx, ebias):
    n_rows, DW = table.shape
    BE, K = idx.shape
    KH = K // 2
    info = plsc.get_sparse_core_info()
    nw = info.num_cores * info.num_subcores
    epw = BE // nw

    mesh = plsc.VectorSubcoreMesh(core_axis_name="c", subcore_axis_name="s")

    @functools.partial(
        pl.kernel,
        mesh=mesh,
        out_type=jax.ShapeDtypeStruct((BE, 2, DW), jnp.float32),
        scratch_types=[
            pltpu.VMEM((epw, K), jnp.int32),
            pltpu.VMEM((epw,), jnp.float32),
            pltpu.VMEM((KH, DW), jnp.int32),
            pltpu.VMEM((KH, DW), jnp.int32),
            pltpu.VMEM((KH, DW), jnp.int32),
            pltpu.VMEM((KH, DW), jnp.int32),
            pltpu.VMEM((epw, 2, DW), jnp.float32),
            pltpu.SemaphoreType.DMA,
            pltpu.SemaphoreType.DMA,
            pltpu.SemaphoreType.DMA,
            pltpu.SemaphoreType.DMA,
        ],
    )
    def run(table_hbm, idx_hbm, ebias_hbm, out_hbm,
            idx_v, ebias_v, buf0, buf1, buf2, buf3, out_v,
            sem0, sem1, sem2, sem3):
        wid = lax.axis_index("s") * info.num_cores + lax.axis_index("c")
        base = wid * epw
        pltpu.sync_copy(idx_hbm.at[pl.ds(base, epw), :], idx_v)
        pltpu.sync_copy(ebias_hbm.at[pl.ds(base, epw)], ebias_v)

        bufs = (buf0, buf1, buf2, buf3)
        sems = (sem0, sem1, sem2, sem3)

        def slot(e, h):
            return (2 * e + h) % 4

        def copy(e, h):
            s = slot(e, h)
            return pltpu.make_async_copy(
                table_hbm.at[idx_v.at[e, pl.ds(h * KH, KH)]], bufs[s], sems[s])

        def split(w):
            a = lax.bitcast_convert_type(w << 16, jnp.float32)
            b = lax.bitcast_convert_type(w & _HI_MASK, jnp.float32)
            return a, b

        for e in (0, 1):
            for h in (0, 1):
                copy(e, h).start()

        for e in range(epw):
            ev = ebias_v[pl.ds((e // L) * L, L)]
            eb = jnp.full((L,), ev[e % L], dtype=jnp.float32)
            for h in (0, 1):
                copy(e, h).wait()
                buf = bufs[slot(e, h)]

                def cbody(c, carry, buf=buf, e=e, h=h, eb=eb):
                    off = c * L
                    if h == 0:
                        acc_a, acc_b = split(buf[0, pl.ds(off, L)])
                        k0 = 1
                    else:
                        acc_a = out_v[e, 0, pl.ds(off, L)]
                        acc_b = out_v[e, 1, pl.ds(off, L)]
                        k0 = 0
                    for kk in range(k0, KH):
                        a, b = split(buf[kk, pl.ds(off, L)])
                        acc_a = jnp.maximum(acc_a, a)
                        acc_b = jnp.maximum(acc_b, b)
                    if h == 1:
                        acc_a = acc_a + eb
                        acc_b = acc_b + eb
                    out_v[e, 0, pl.ds(off, L)] = acc_a
                    out_v[e, 1, pl.ds(off, L)] = acc_b
                    return carry

                lax.fori_loop(0, DW // L, cbody, 0)
                if e + 2 < epw:
                    copy(e + 2, h).start()
        pltpu.sync_copy(out_v, out_hbm.at[pl.ds(base, epw), :, :])

    return run(table, idx, ebias)


def kernel(mention_reprs, entities, entity_masks):
    B, M, D = mention_reprs.shape
    _, E, K = entities.shape
    table = lax.bitcast_convert_type(
        mention_reprs.astype(jnp.bfloat16).reshape(B * M, D // 2, 2),
        jnp.int32)
    keep = entity_masks != 0
    any_kept = jnp.any(keep, axis=-1, keepdims=True)
    first = jnp.argmax(keep, axis=-1)
    fill = jnp.take_along_axis(entities, first[..., None], axis=-1)
    idx_eff = jnp.where(keep | ~any_kept, entities, fill)
    idx = (idx_eff + (jnp.arange(B, dtype=jnp.int32) * M)[:, None, None]
           ).reshape(B * E, K)
    ebias = jnp.where(any_kept[..., 0], jnp.float32(0), jnp.float32(-1e30)
                      ).reshape(B * E)
    out = _entity_pool_sc(table, idx, ebias)
    out = out.transpose(0, 2, 1).reshape(B, E, D)
    return out


# speedup vs baseline: 2.2648x; 2.2648x over previous
import functools

import jax
import jax.numpy as jnp
from jax import lax
from jax.experimental import pallas as pl
from jax.experimental.pallas import tpu as pltpu
from jax.experimental.pallas import tpu_sc as plsc

L = 16


def _entity_pool_sc(table, idx, masks, M):
    n_rows, D = table.shape
    BE, K = idx.shape
    KH = K // 2
    info = plsc.get_sparse_core_info()
    nw = info.num_cores * info.num_subcores
    epw = BE // nw

    mesh = plsc.VectorSubcoreMesh(core_axis_name="c", subcore_axis_name="s")

    @functools.partial(
        pl.kernel,
        mesh=mesh,
        out_type=jax.ShapeDtypeStruct((BE, D), jnp.float32),
        scratch_types=[
            pltpu.VMEM((epw, K), jnp.int32),
            pltpu.VMEM((epw, K), jnp.int32),
            pltpu.VMEM((KH, D), jnp.float32),
            pltpu.VMEM((KH, D), jnp.float32),
            pltpu.VMEM((KH, D), jnp.float32),
            pltpu.VMEM((KH, D), jnp.float32),
            pltpu.VMEM((epw, D), jnp.float32),
            pltpu.SemaphoreType.DMA,
            pltpu.SemaphoreType.DMA,
            pltpu.SemaphoreType.DMA,
            pltpu.SemaphoreType.DMA,
        ],
    )
    def run(table_hbm, idx_hbm, mask_hbm, out_hbm,
            idx_v, mask_v, buf0, buf1, buf2, buf3, out_v,
            sem0, sem1, sem2, sem3):
        wid = lax.axis_index("s") * info.num_cores + lax.axis_index("c")
        base = wid * epw
        pltpu.sync_copy(idx_hbm.at[pl.ds(base, epw), :], idx_v)
        pltpu.sync_copy(mask_hbm.at[pl.ds(base, epw), :], mask_v)

        boff = jnp.full((L,), (base // (BE // (n_rows // M))) * M,
                        dtype=jnp.int32)
        for e in range(epw):
            for h in range(K // L):
                sl = pl.ds(h * L, L)
                idx_v[e, sl] = idx_v[e, sl] + boff

        bufs = (buf0, buf1, buf2, buf3)
        sems = (sem0, sem1, sem2, sem3)

        def slot(e, h):
            return (2 * e + h) % 4

        def copy(e, h):
            s = slot(e, h)
            return pltpu.make_async_copy(
                table_hbm.at[idx_v.at[e, pl.ds(h * KH, KH)]], bufs[s], sems[s])

        for e in (0, 1):
            for h in (0, 1):
                copy(e, h).start()

        for e in range(epw):
            for h in (0, 1):
                copy(e, h).wait()
                buf = bufs[slot(e, h)]
                mv = mask_v[e, pl.ds(h * KH, L)]
                bv = jnp.where(mv == 0, jnp.float32(-1e30), jnp.float32(0.0))
                splats = [jnp.full((L,), bv[j], dtype=jnp.float32)
                          for j in range(KH)]

                def cbody(c, carry, buf=buf, splats=splats, e=e, h=h):
                    off = c * L
                    if h == 0:
                        acc = buf[0, pl.ds(off, L)] + splats[0]
                        k0 = 1
                    else:
                        acc = out_v[e, pl.ds(off, L)]
                        k0 = 0
                    for kk in range(k0, KH):
                        acc = jnp.maximum(acc, buf[kk, pl.ds(off, L)] + splats[kk])
                    out_v[e, pl.ds(off, L)] = acc
                    return carry

                lax.fori_loop(0, D // L, cbody, 0)
                if e + 2 < epw:
                    copy(e + 2, h).start()
        pltpu.sync_copy(out_v, out_hbm.at[pl.ds(base, epw), :])

    return run(table, idx, masks)


def kernel(mention_reprs, entities, entity_masks):
    B, M, D = mention_reprs.shape
    _, E, K = entities.shape
    table = mention_reprs.reshape(B * M, D)
    idx = entities.reshape(B * E, K)
    masks = entity_masks.reshape(B * E, K)
    out = _entity_pool_sc(table, idx, masks, M)
    return out.reshape(B, E, D)
